# baseline (speedup 1.0000x reference)
import functools

import jax
import jax.numpy as jnp
from jax import lax
from jax.experimental import pallas as pl
from jax.experimental.pallas import tpu as pltpu
from jax.experimental.pallas import tpu_sc as plsc

EMSIZE = 64
N_ROWS = 4096
N_COLS = 200
B_FLAT = N_ROWS * N_COLS

_INFO = plsc.get_sparse_core_info()
NW = _INFO.num_cores * _INFO.num_subcores
ROWS_W = N_ROWS // NW
PER_W = ROWS_W * N_COLS
CHUNK = N_COLS
NCHUNK = ROWS_W
NBUF = 4
NITER = NCHUNK // NBUF


def _sc_gather(idx_flat, table):
    mesh = plsc.VectorSubcoreMesh(core_axis_name="c", subcore_axis_name="s")

    @functools.partial(
        pl.kernel,
        mesh=mesh,
        out_type=jax.ShapeDtypeStruct((N_ROWS, N_COLS, EMSIZE), jnp.float32),
        compiler_params=pltpu.CompilerParams(use_tc_tiling_on_sc=False),
        scratch_types=[
            pltpu.VMEM((PER_W,), jnp.int32),
            pltpu.VMEM((NBUF, N_COLS, EMSIZE), jnp.float32),
            pltpu.SemaphoreType.DMA((NBUF,)),
            pltpu.SemaphoreType.DMA((NBUF,)),
        ],
    )
    def body(idx_hbm, table_hbm, out_hbm, idx_v, rows_v, gsem, wsem):
        wid = lax.axis_index("s") * _INFO.num_cores + lax.axis_index("c")
        base = wid * PER_W
        row0 = wid * ROWS_W
        pltpu.sync_copy(idx_hbm.at[pl.ds(base, PER_W)], idx_v)

        def gather_start(g, b):
            pltpu.async_copy(
                table_hbm.at[idx_v.at[pl.ds(g * CHUNK, CHUNK)]],
                rows_v.at[b],
                gsem.at[b],
            )

        def gather_wait(b):
            pltpu.make_async_copy(
                table_hbm.at[idx_v.at[pl.ds(0, CHUNK)]],
                rows_v.at[b],
                gsem.at[b],
            ).wait()

        def write_start(g, b):
            pltpu.async_copy(
                rows_v.at[b],
                out_hbm.at[row0 + g],
                wsem.at[b],
            )

        def write_wait(b):
            pltpu.make_async_copy(
                rows_v.at[b],
                out_hbm.at[row0],
                wsem.at[b],
            ).wait()

        for b in range(NBUF):
            gather_start(b, b)

        def loop_body(it, _):
            g0 = it * NBUF
            for b in range(NBUF):
                g = g0 + b
                gather_wait(b)
                write_start(g, b)

                @pl.when(g + NBUF < NCHUNK)
                def _():
                    write_wait(b)
                    gather_start(g + NBUF, b)

            return ()

        lax.fori_loop(0, NITER, loop_body, ())

        for b in range(NBUF):
            write_wait(b)

    return body(idx_flat, table)


@jax.jit
def kernel(input, table):
    idx_flat = jnp.reshape(input, (B_FLAT,)).astype(jnp.int32)
    return _sc_gather(idx_flat, table)


# speedup vs baseline: 1.2172x; 1.2172x over previous
import functools

import jax
import jax.numpy as jnp
from jax import lax
from jax.experimental import pallas as pl
from jax.experimental.pallas import tpu as pltpu
from jax.experimental.pallas import tpu_sc as plsc

EMSIZE = 64
PAD = 128
N_ROWS = 4096
N_COLS = 200
B_FLAT = N_ROWS * N_COLS

_INFO = plsc.get_sparse_core_info()
NW = _INFO.num_cores * _INFO.num_subcores
ROWS_W = N_ROWS // NW
PER_W = ROWS_W * N_COLS
CHUNK = N_COLS
NCHUNK = ROWS_W
NBUF = 4
NITER = NCHUNK // NBUF


def _sc_gather(idx_flat, table_pad):
    mesh = plsc.VectorSubcoreMesh(core_axis_name="c", subcore_axis_name="s")

    @functools.partial(
        pl.kernel,
        mesh=mesh,
        out_type=jax.ShapeDtypeStruct((N_ROWS, N_COLS, PAD), jnp.float32),
        compiler_params=pltpu.CompilerParams(use_tc_tiling_on_sc=True),
        scratch_types=[
            pltpu.VMEM((PER_W,), jnp.int32),
            pltpu.VMEM((NBUF, N_COLS, PAD), jnp.float32),
            pltpu.SemaphoreType.DMA((NBUF,)),
            pltpu.SemaphoreType.DMA((NBUF,)),
        ],
    )
    def body(idx_hbm, table_hbm, out_hbm, idx_v, rows_v, gsem, wsem):
        wid = lax.axis_index("s") * _INFO.num_cores + lax.axis_index("c")
        base = wid * PER_W
        row0 = wid * ROWS_W
        pltpu.sync_copy(idx_hbm.at[pl.ds(base, PER_W)], idx_v)

        def gather_start(g, b):
            pltpu.async_copy(
                table_hbm.at[idx_v.at[pl.ds(g * CHUNK, CHUNK)]],
                rows_v.at[b],
                gsem.at[b],
            )

        def gather_wait(b):
            pltpu.make_async_copy(
                table_hbm.at[idx_v.at[pl.ds(0, CHUNK)]],
                rows_v.at[b],
                gsem.at[b],
            ).wait()

        def write_start(g, b):
            pltpu.async_copy(
                rows_v.at[b],
                out_hbm.at[row0 + g],
                wsem.at[b],
            )

        def write_wait(b):
            pltpu.make_async_copy(
                rows_v.at[b],
                out_hbm.at[row0],
                wsem.at[b],
            ).wait()

        for b in range(NBUF):
            gather_start(b, b)

        def loop_body(it, _):
            g0 = it * NBUF
            for b in range(NBUF):
                g = g0 + b
                gather_wait(b)
                write_start(g, b)

                @pl.when(g + NBUF < NCHUNK)
                def _():
                    write_wait(b)
                    gather_start(g + NBUF, b)

            return ()

        lax.fori_loop(0, NITER, loop_body, ())

        for b in range(NBUF):
            write_wait(b)

    return body(idx_flat, table_pad)


@jax.jit
def kernel(input, table):
    idx_flat = jnp.reshape(input, (B_FLAT,)).astype(jnp.int32)
    table_pad = jnp.pad(table, ((0, 0), (0, PAD - EMSIZE)))
    out128 = _sc_gather(idx_flat, table_pad)
    return out128[:, :, :EMSIZE]


# speedup vs baseline: 1.2191x; 1.0015x over previous
import functools

import jax
import jax.numpy as jnp
from jax import lax
from jax.experimental import pallas as pl
from jax.experimental.pallas import tpu as pltpu
from jax.experimental.pallas import tpu_sc as plsc

EMSIZE = 64
N_ROWS = 4096
N_COLS = 200
B_FLAT = N_ROWS * N_COLS

_INFO = plsc.get_sparse_core_info()
NW = _INFO.num_cores * _INFO.num_subcores
ROWS_W = N_ROWS // NW
PER_W = ROWS_W * N_COLS
CHUNK = N_COLS
NCHUNK = ROWS_W
NBUF = 4
NITER = NCHUNK // NBUF


def _sc_gather(idx_flat, table):
    mesh = plsc.VectorSubcoreMesh(core_axis_name="c", subcore_axis_name="s")

    @functools.partial(
        pl.kernel,
        mesh=mesh,
        out_type=jax.ShapeDtypeStruct((N_ROWS, N_COLS, 128), jnp.float32),
        compiler_params=pltpu.CompilerParams(use_tc_tiling_on_sc=True),
        scratch_types=[
            pltpu.VMEM((PER_W,), jnp.int32),
            pltpu.VMEM((NBUF, N_COLS, 128), jnp.float32),
            pltpu.SemaphoreType.DMA((NBUF,)),
            pltpu.SemaphoreType.DMA((NBUF,)),
        ],
    )
    def body(idx_hbm, table_hbm, out_hbm, idx_v, rows_v, gsem, wsem):
        wid = lax.axis_index("s") * _INFO.num_cores + lax.axis_index("c")
        base = wid * PER_W
        row0 = wid * ROWS_W
        pltpu.sync_copy(idx_hbm.at[pl.ds(base, PER_W)], idx_v)

        def gather_start(g, b):
            pltpu.async_copy(
                table_hbm.at[idx_v.at[pl.ds(g * CHUNK, CHUNK)]],
                rows_v.at[b],
                gsem.at[b],
            )

        def gather_wait(b):
            pltpu.make_async_copy(
                table_hbm.at[idx_v.at[pl.ds(0, CHUNK)]],
                rows_v.at[b],
                gsem.at[b],
            ).wait()

        def write_start(g, b):
            pltpu.async_copy(
                rows_v.at[b],
                out_hbm.at[row0 + g],
                wsem.at[b],
            )

        def write_wait(b):
            pltpu.make_async_copy(
                rows_v.at[b],
                out_hbm.at[row0],
                wsem.at[b],
            ).wait()

        for b in range(NBUF):
            gather_start(b, b)

        def loop_body(it, _):
            g0 = it * NBUF
            for b in range(NBUF):
                g = g0 + b
                gather_wait(b)
                write_start(g, b)

                @pl.when(g + NBUF < NCHUNK)
                def _():
                    write_wait(b)
                    gather_start(g + NBUF, b)

            return ()

        lax.fori_loop(0, NITER, loop_body, ())

        for b in range(NBUF):
            write_wait(b)

    return body(idx_flat, table)


@jax.jit
def kernel(input, table):
    idx_flat = jnp.reshape(input, (B_FLAT,)).astype(jnp.int32)
    table_pad = jnp.pad(table, ((0, 0), (0, 128 - EMSIZE)))
    out128 = _sc_gather(idx_flat, table_pad)
    return out128[:, :, :EMSIZE]
